# baseline (speedup 1.0000x reference)
import functools

import jax
import jax.numpy as jnp
from jax import lax
from jax.experimental import pallas as pl
from jax.experimental.pallas import tpu as pltpu
from jax.experimental.pallas import tpu_sc as plsc

_B, _T, _D = 4, 4096, 2048
_N = _B * _T
_NE = 16
_NSCORE = 24
_THRESH = 0.15
_NEG = -1e30
_NW = 32
_CHUNK = _N // _NW
_NG = _CHUNK // 16


def _scores_body(x_ref, w_ref, st_ref):
    s = jax.lax.dot_general(
        x_ref[...], w_ref[...], (((1,), (0,)), ((), ())),
        preferred_element_type=jnp.float32)
    st_ref[...] = s.T


def _route_body(st_hbm, rw_hbm, idx_hbm, cnt_hbm, sv, rwv, idxv, cntv):
    wid = lax.axis_index("s") * 2 + lax.axis_index("c")
    base = wid * _CHUNK
    pltpu.sync_copy(st_hbm.at[:, pl.ds(base, _CHUNK)], sv)
    cntv[...] = jnp.zeros((16,), jnp.float32)
    iota = lax.iota(jnp.int32, 16)
    zf = jnp.zeros((16,), jnp.float32)
    zi = jnp.zeros((16,), jnp.int32)
    ones = jnp.ones((16,), jnp.float32)

    def group(g, pos):
        col = g * 16
        s = [sv[e, pl.ds(col, 16)] for e in range(18)]
        sig0 = 1.0 / (1.0 + jnp.exp(-s[16]))
        sig1 = 1.0 / (1.0 + jnp.exp(-s[17]))

        def top1(lo, hi):
            m = s[lo]
            for e in range(lo + 1, hi):
                m = jnp.maximum(m, s[e])
            idx = jnp.full((16,), hi - 1, jnp.int32)
            for e in range(hi - 2, lo - 1, -1):
                idx = jnp.where(s[e] == m, e, idx)
            z = zf
            for e in range(lo, hi):
                z = z + jnp.exp(s[e] - m)
            return m, idx, z

        _, idx_a, z_a = top1(0, 8)
        w0 = 1.0 / z_a

        _, idx_b, z_b = top1(8, 12)
        w1 = (1.0 / z_b) * sig0 * (sig0 > _THRESH).astype(jnp.float32)

        m_c, idx_c1, z_c = top1(12, 16)
        s2 = [jnp.where(idx_c1 == e, _NEG, s[e]) for e in range(12, 16)]
        m2 = jnp.maximum(jnp.maximum(s2[0], s2[1]), jnp.maximum(s2[2], s2[3]))
        idx_c2 = jnp.full((16,), 15, jnp.int32)
        for e in range(14, 11, -1):
            idx_c2 = jnp.where(s2[e - 12] == m2, e, idx_c2)
        gate_c = sig1 * (sig1 > _THRESH).astype(jnp.float32)
        w2 = (1.0 / z_c) * gate_c
        w3 = (jnp.exp(m2 - m_c) / z_c) * gate_c

        rnorm = 1.0 / (w0 + w1 + w2 + w3 + 1e-8)
        plsc.store_scatter(rwv, [pos], w0 * rnorm)
        plsc.store_scatter(rwv, [pos + 1], w1 * rnorm)
        plsc.store_scatter(rwv, [pos + 2], w2 * rnorm)
        plsc.store_scatter(rwv, [pos + 3], w3 * rnorm)
        plsc.store_scatter(rwv, [pos + 4], zf)
        plsc.store_scatter(rwv, [pos + 5], zf)
        plsc.store_scatter(idxv, [pos], idx_a)
        plsc.store_scatter(idxv, [pos + 1], idx_b)
        plsc.store_scatter(idxv, [pos + 2], idx_c1)
        plsc.store_scatter(idxv, [pos + 3], idx_c2)
        plsc.store_scatter(idxv, [pos + 4], zi)
        plsc.store_scatter(idxv, [pos + 5], zi)
        plsc.addupdate_scatter(cntv, [idx_a], ones)
        plsc.addupdate_scatter(cntv, [idx_b], ones)
        plsc.addupdate_scatter(cntv, [idx_c1], ones)
        plsc.addupdate_scatter(cntv, [idx_c2], ones)
        return pos + 96

    lax.fori_loop(0, _NG, group, iota * 6)
    pltpu.sync_copy(rwv, rw_hbm.at[pl.ds(base * 6, _CHUNK * 6)])
    pltpu.sync_copy(idxv, idx_hbm.at[pl.ds(base * 6, _CHUNK * 6)])
    pltpu.sync_copy(cntv, cnt_hbm.at[wid])


def _aux_body(cnt_ref, aux_ref):
    c = jnp.sum(cnt_ref[...], axis=0, keepdims=True)
    lane = lax.broadcasted_iota(jnp.int32, (1, _NE), 1)
    c = c + jnp.where(lane == 0, jnp.float32(2 * _B * _T), 0.0)
    total = jnp.sum(c)
    aux = (0.01 / _NE) * jnp.sum(
        jnp.log(jnp.float32(1.0 / _NE)) - jnp.log(c / total),
        axis=-1, keepdims=True)
    aux_ref[...] = aux


@functools.partial(jax.jit, static_argnames=("tb",))
def _run(x, W_expert, W_group, tb=2048):
    nblocks = _N // tb
    xf = x.reshape(_N, _D)
    w18 = jnp.concatenate([W_expert, W_group], axis=0)
    wt = jnp.pad(w18, ((0, _NSCORE - 18), (0, 0))).T

    scores_t = pl.pallas_call(
        _scores_body,
        grid=(nblocks,),
        in_specs=[
            pl.BlockSpec((tb, _D), lambda i: (i, 0)),
            pl.BlockSpec((_D, _NSCORE), lambda i: (0, 0)),
        ],
        out_specs=pl.BlockSpec((_NSCORE, tb), lambda i: (0, i)),
        out_shape=jax.ShapeDtypeStruct((_NSCORE, _N), jnp.float32),
        compiler_params=pltpu.CompilerParams(
            dimension_semantics=("arbitrary",)),
    )(xf, wt)

    route = functools.partial(
        pl.kernel,
        out_type=[
            jax.ShapeDtypeStruct((_N * 6,), jnp.float32),
            jax.ShapeDtypeStruct((_N * 6,), jnp.int32),
            jax.ShapeDtypeStruct((_NW, _NE), jnp.float32),
        ],
        mesh=plsc.VectorSubcoreMesh(core_axis_name="c", subcore_axis_name="s"),
        scratch_types=[
            pltpu.VMEM((_NSCORE, _CHUNK), jnp.float32),
            pltpu.VMEM((_CHUNK * 6,), jnp.float32),
            pltpu.VMEM((_CHUNK * 6,), jnp.int32),
            pltpu.VMEM((16,), jnp.float32),
        ],
        compiler_params=pltpu.CompilerParams(needs_layout_passes=False),
    )(_route_body)
    rw_flat, idx_flat, counts = route(scores_t)

    aux = pl.pallas_call(
        _aux_body,
        out_shape=jax.ShapeDtypeStruct((1, 1), jnp.float32),
    )(counts)

    return (rw_flat.reshape(_B, _T, 6), idx_flat.reshape(_B, _T, 6),
            aux[0, 0])


def kernel(x, W_expert, W_group):
    return _run(x, W_expert, W_group)


# speedup vs baseline: 1.0194x; 1.0194x over previous
import functools

import jax
import jax.numpy as jnp
from jax import lax
from jax.experimental import pallas as pl
from jax.experimental.pallas import tpu as pltpu
from jax.experimental.pallas import tpu_sc as plsc

_B, _T, _D = 4, 4096, 2048
_N = _B * _T
_NE = 16
_NSCORE = 24
_THRESH = 0.15
_NEG = -1e30
_NW = 32
_CHUNK = _N // _NW
_NG = _CHUNK // 16


def _scores_body(x_ref, w_ref, st_ref):
    s = jax.lax.dot_general(
        x_ref[...].astype(jnp.bfloat16), w_ref[...].astype(jnp.bfloat16),
        (((1,), (0,)), ((), ())),
        preferred_element_type=jnp.float32)
    st_ref[...] = s.T


def _route_body(st_hbm, rw_hbm, idx_hbm, cnt_hbm, sv, rwv, idxv, cntv):
    wid = lax.axis_index("s") * 2 + lax.axis_index("c")
    base = wid * _CHUNK
    pltpu.sync_copy(st_hbm.at[:, pl.ds(base, _CHUNK)], sv)
    cntv[...] = jnp.zeros((16,), jnp.float32)
    iota = lax.iota(jnp.int32, 16)
    zf = jnp.zeros((16,), jnp.float32)
    zi = jnp.zeros((16,), jnp.int32)
    ones = jnp.ones((16,), jnp.float32)

    def group(g, pos):
        col = g * 16
        s = [sv[e, pl.ds(col, 16)] for e in range(18)]
        sig0 = 1.0 / (1.0 + jnp.exp(-s[16]))
        sig1 = 1.0 / (1.0 + jnp.exp(-s[17]))

        def top1(lo, hi):
            m = s[lo]
            for e in range(lo + 1, hi):
                m = jnp.maximum(m, s[e])
            idx = jnp.full((16,), hi - 1, jnp.int32)
            for e in range(hi - 2, lo - 1, -1):
                idx = jnp.where(s[e] == m, e, idx)
            z = zf
            for e in range(lo, hi):
                z = z + jnp.exp(s[e] - m)
            return m, idx, z

        _, idx_a, z_a = top1(0, 8)
        w0 = 1.0 / z_a

        _, idx_b, z_b = top1(8, 12)
        w1 = (1.0 / z_b) * sig0 * (sig0 > _THRESH).astype(jnp.float32)

        m_c, idx_c1, z_c = top1(12, 16)
        s2 = [jnp.where(idx_c1 == e, _NEG, s[e]) for e in range(12, 16)]
        m2 = jnp.maximum(jnp.maximum(s2[0], s2[1]), jnp.maximum(s2[2], s2[3]))
        idx_c2 = jnp.full((16,), 15, jnp.int32)
        for e in range(14, 11, -1):
            idx_c2 = jnp.where(s2[e - 12] == m2, e, idx_c2)
        gate_c = sig1 * (sig1 > _THRESH).astype(jnp.float32)
        w2 = (1.0 / z_c) * gate_c
        w3 = (jnp.exp(m2 - m_c) / z_c) * gate_c

        rnorm = 1.0 / (w0 + w1 + w2 + w3 + 1e-8)
        plsc.store_scatter(rwv, [pos], w0 * rnorm)
        plsc.store_scatter(rwv, [pos + 1], w1 * rnorm)
        plsc.store_scatter(rwv, [pos + 2], w2 * rnorm)
        plsc.store_scatter(rwv, [pos + 3], w3 * rnorm)
        plsc.store_scatter(rwv, [pos + 4], zf)
        plsc.store_scatter(rwv, [pos + 5], zf)
        plsc.store_scatter(idxv, [pos], idx_a)
        plsc.store_scatter(idxv, [pos + 1], idx_b)
        plsc.store_scatter(idxv, [pos + 2], idx_c1)
        plsc.store_scatter(idxv, [pos + 3], idx_c2)
        plsc.store_scatter(idxv, [pos + 4], zi)
        plsc.store_scatter(idxv, [pos + 5], zi)
        plsc.addupdate_scatter(cntv, [idx_a], ones)
        plsc.addupdate_scatter(cntv, [idx_b], ones)
        plsc.addupdate_scatter(cntv, [idx_c1], ones)
        plsc.addupdate_scatter(cntv, [idx_c2], ones)
        return pos + 96

    lax.fori_loop(0, _NG, group, iota * 6)
    pltpu.sync_copy(rwv, rw_hbm.at[pl.ds(base * 6, _CHUNK * 6)])
    pltpu.sync_copy(idxv, idx_hbm.at[pl.ds(base * 6, _CHUNK * 6)])
    pltpu.sync_copy(cntv, cnt_hbm.at[wid])


def _aux_body(cnt_ref, aux_ref):
    c = jnp.sum(cnt_ref[...], axis=0, keepdims=True)
    lane = lax.broadcasted_iota(jnp.int32, (1, _NE), 1)
    c = c + jnp.where(lane == 0, jnp.float32(2 * _B * _T), 0.0)
    total = jnp.sum(c)
    aux = (0.01 / _NE) * jnp.sum(
        jnp.log(jnp.float32(1.0 / _NE)) - jnp.log(c / total),
        axis=-1, keepdims=True)
    aux_ref[...] = aux


@functools.partial(jax.jit, static_argnames=("tb",))
def _run(x, W_expert, W_group, tb=1024):
    nblocks = _N // tb
    xf = x.reshape(_N, _D)
    w18 = jnp.concatenate([W_expert, W_group], axis=0)
    wt = jnp.pad(w18, ((0, _NSCORE - 18), (0, 0))).T

    scores_t = pl.pallas_call(
        _scores_body,
        grid=(nblocks,),
        in_specs=[
            pl.BlockSpec((tb, _D), lambda i: (i, 0)),
            pl.BlockSpec((_D, _NSCORE), lambda i: (0, 0)),
        ],
        out_specs=pl.BlockSpec((_NSCORE, tb), lambda i: (0, i)),
        out_shape=jax.ShapeDtypeStruct((_NSCORE, _N), jnp.float32),
        compiler_params=pltpu.CompilerParams(
            dimension_semantics=("arbitrary",)),
    )(xf, wt)

    route = functools.partial(
        pl.kernel,
        out_type=[
            jax.ShapeDtypeStruct((_N * 6,), jnp.float32),
            jax.ShapeDtypeStruct((_N * 6,), jnp.int32),
            jax.ShapeDtypeStruct((_NW, _NE), jnp.float32),
        ],
        mesh=plsc.VectorSubcoreMesh(core_axis_name="c", subcore_axis_name="s"),
        scratch_types=[
            pltpu.VMEM((_NSCORE, _CHUNK), jnp.float32),
            pltpu.VMEM((_CHUNK * 6,), jnp.float32),
            pltpu.VMEM((_CHUNK * 6,), jnp.int32),
            pltpu.VMEM((16,), jnp.float32),
        ],
        compiler_params=pltpu.CompilerParams(needs_layout_passes=False),
    )(_route_body)
    rw_flat, idx_flat, counts = route(scores_t)

    aux = pl.pallas_call(
        _aux_body,
        out_shape=jax.ShapeDtypeStruct((1, 1), jnp.float32),
    )(counts)

    return (rw_flat.reshape(_B, _T, 6), idx_flat.reshape(_B, _T, 6),
            aux[0, 0])


def kernel(x, W_expert, W_group):
    return _run(x, W_expert, W_group)


# speedup vs baseline: 1.2278x; 1.2045x over previous
import functools

import jax
import jax.numpy as jnp
from jax.experimental import pallas as pl
from jax.experimental.pallas import tpu as pltpu

_B, _T, _D = 4, 4096, 2048
_NE = 16
_THRESH = 0.15
_NEG = -1e30


def _router_body(x_ref, w_ref, rw_ref, idx_ref, aux_ref, cnt_ref, *, nblocks, tb):
    i = pl.program_id(0)

    xb = x_ref[...]
    scores = jax.lax.dot_general(
        xb.astype(jnp.bfloat16), w_ref[...].astype(jnp.bfloat16),
        (((1,), (0,)), ((), ())),
        preferred_element_type=jnp.float32)
    es = scores[:, :_NE]
    g0 = jax.nn.sigmoid(scores[:, _NE:_NE + 1])
    g1 = jax.nn.sigmoid(scores[:, _NE + 1:_NE + 2])

    lane = jax.lax.broadcasted_iota(jnp.int32, (tb, _NE), 1)
    mask_a = lane < 8
    mask_b = jnp.logical_and(lane >= 8, lane < 12)
    mask_c = lane >= 12

    def top1(mask, s):
        sm = jnp.where(mask, s, _NEG)
        m = jnp.max(sm, axis=-1, keepdims=True)
        idx = jnp.min(jnp.where(sm == m, lane, _NE), axis=-1, keepdims=True)
        z = jnp.sum(jnp.where(mask, jnp.exp(s - m), 0.0), axis=-1, keepdims=True)
        return m, idx, z

    m_a, idx_a, z_a = top1(mask_a, es)
    p_a = 1.0 / z_a

    m_b, idx_b, z_b = top1(mask_b, es)
    w_b = (1.0 / z_b) * g0 * (g0 > _THRESH).astype(jnp.float32)

    m_c, idx_c1, z_c = top1(mask_c, es)
    p_c1 = 1.0 / z_c
    mask_c2 = jnp.logical_and(mask_c, lane != idx_c1)
    sm2 = jnp.where(mask_c2, es, _NEG)
    m_c2 = jnp.max(sm2, axis=-1, keepdims=True)
    idx_c2 = jnp.min(jnp.where(sm2 == m_c2, lane, _NE), axis=-1, keepdims=True)
    p_c2 = jnp.exp(m_c2 - m_c) / z_c
    gate_c = g1 * (g1 > _THRESH).astype(jnp.float32)
    w_c1 = p_c1 * gate_c
    w_c2 = p_c2 * gate_c

    zeros = jnp.zeros((tb, 2), jnp.float32)
    rw = jnp.concatenate([p_a, w_b, w_c1, w_c2, zeros], axis=-1)
    rw = rw / (jnp.sum(rw, axis=-1, keepdims=True) + 1e-8)
    rw_ref[...] = rw
    izeros = jnp.zeros((tb, 2), jnp.int32)
    idx_ref[...] = jnp.concatenate([idx_a, idx_b, idx_c1, idx_c2, izeros],
                                   axis=-1)

    bc = jnp.zeros((1, _NE), jnp.float32)
    for idx in (idx_a, idx_b, idx_c1, idx_c2):
        oh = (jnp.broadcast_to(idx, (tb, _NE)) == lane).astype(jnp.float32)
        bc = bc + jnp.sum(oh, axis=0, keepdims=True)

    @pl.when(i == 0)
    def _():
        cnt_ref[...] = jnp.zeros_like(cnt_ref)

    cnt_ref[0:1, 0:_NE] += bc

    @pl.when(i == nblocks - 1)
    def _():
        lane1 = jax.lax.broadcasted_iota(jnp.int32, (1, _NE), 1)
        pad = jnp.where(lane1 == 0, jnp.float32(2 * _B * _T), 0.0)
        counts = cnt_ref[0:1, 0:_NE] + pad
        total = jnp.sum(counts)
        log_u = jnp.log(jnp.float32(1.0 / _NE))
        aux = (0.01 / _NE) * jnp.sum(log_u - jnp.log(counts / total),
                                     axis=-1, keepdims=True)
        aux_ref[...] = aux


@functools.partial(jax.jit, static_argnames=("tb",))
def _run(x, W_expert, W_group, tb=1024):
    n = _B * _T
    nblocks = n // tb
    xf = x.reshape(n, _D)
    w = jnp.concatenate([W_expert, W_group], axis=0).T

    rw, idx, aux = pl.pallas_call(
        functools.partial(_router_body, nblocks=nblocks, tb=tb),
        grid=(nblocks,),
        in_specs=[
            pl.BlockSpec((tb, _D), lambda i: (i, 0)),
            pl.BlockSpec((_D, _NE + 2), lambda i: (0, 0)),
        ],
        out_specs=[
            pl.BlockSpec((tb, 6), lambda i: (i, 0)),
            pl.BlockSpec((tb, 6), lambda i: (i, 0)),
            pl.BlockSpec((1, 1), lambda i: (0, 0)),
        ],
        out_shape=[
            jax.ShapeDtypeStruct((n, 6), jnp.float32),
            jax.ShapeDtypeStruct((n, 6), jnp.int32),
            jax.ShapeDtypeStruct((1, 1), jnp.float32),
        ],
        scratch_shapes=[pltpu.VMEM((8, 128), jnp.float32)],
        compiler_params=pltpu.CompilerParams(
            dimension_semantics=("arbitrary",)),
    )(xf, w)

    return (rw.reshape(_B, _T, 6), idx.reshape(_B, _T, 6), aux[0, 0])


def kernel(x, W_expert, W_group):
    return _run(x, W_expert, W_group)
